# baseline (speedup 1.0000x reference)
import functools

import jax
import jax.numpy as jnp
from jax import lax
from jax.experimental import pallas as pl
from jax.experimental.pallas import tpu as pltpu
from jax.experimental.pallas import tpu_sc as plsc

BATCH = 16384
EMBED_DIM = 128

_info = plsc.get_sparse_core_info()
_NC, _NS = _info.num_cores, _info.num_subcores
_NW = _NC * _NS
_B_PER_W = BATCH // _NW
_N_CHUNK = 4
_CHUNK = _B_PER_W // _N_CHUNK


def _make_gather():
    mesh = plsc.VectorSubcoreMesh(core_axis_name="c", subcore_axis_name="s")

    @functools.partial(
        pl.kernel,
        mesh=mesh,
        out_type=jax.ShapeDtypeStruct((_NW, _N_CHUNK, _CHUNK, EMBED_DIM), jnp.float32),
        scratch_types=[
            pltpu.VMEM((_N_CHUNK, _CHUNK), jnp.int32),
            pltpu.VMEM((_N_CHUNK, _CHUNK, EMBED_DIM), jnp.float32),
            pltpu.VMEM_SHARED((_NS, 2, _CHUNK, EMBED_DIM), jnp.float32),
        ]
        + [pltpu.SemaphoreType.DMA] * (_N_CHUNK + 3),
    )
    def gather_kernel(table_hbm, idx_hbm, out_hbm, idx_v, rows_v, rows_sp, *sems):
        gsems, ssem = sems[:_N_CHUNK], sems[_N_CHUNK]
        osems = sems[_N_CHUNK + 1 :]
        sid = lax.axis_index("s")
        wid = sid * _NC + lax.axis_index("c")
        pltpu.sync_copy(idx_hbm.at[wid], idx_v)
        gathers = [
            pltpu.async_copy(table_hbm.at[idx_v.at[i]], rows_v.at[i], gsems[i])
            for i in range(_N_CHUNK)
        ]
        writes = [None, None]
        for i in range(_N_CHUNK):
            slot = i % 2
            gathers[i].wait()
            if writes[slot] is not None:
                writes[slot].wait()
            pltpu.async_copy(rows_v.at[i], rows_sp.at[sid, slot], ssem).wait()
            writes[slot] = pltpu.async_copy(
                rows_sp.at[sid, slot], out_hbm.at[wid, i], osems[slot]
            )
        for w in writes:
            w.wait()

    return gather_kernel


_gather = _make_gather()


def kernel(t, pos_encoding):
    idx = t.reshape(_NW, _N_CHUNK, _CHUNK).astype(jnp.int32)
    out = _gather(pos_encoding, idx)
    return out.reshape(BATCH, EMBED_DIM)


# speedup vs baseline: 1.1047x; 1.1047x over previous
import functools

import jax
import jax.numpy as jnp
from jax import lax
from jax.experimental import pallas as pl
from jax.experimental.pallas import tpu as pltpu
from jax.experimental.pallas import tpu_sc as plsc

BATCH = 16384
EMBED_DIM = 128

_info = plsc.get_sparse_core_info()
_NC, _NS = _info.num_cores, _info.num_subcores
_NW = _NC * _NS
_B_PER_W = BATCH // _NW


def _make_gather():
    mesh = plsc.VectorSubcoreMesh(core_axis_name="c", subcore_axis_name="s")

    @functools.partial(
        pl.kernel,
        mesh=mesh,
        out_type=jax.ShapeDtypeStruct((BATCH, EMBED_DIM), jnp.float32),
        scratch_types=[
            pltpu.VMEM((_B_PER_W,), jnp.int32),
            pltpu.VMEM((_B_PER_W, EMBED_DIM), jnp.float32),
            pltpu.SemaphoreType.DMA,
        ],
    )
    def gather_kernel(table_hbm, idx_hbm, out_hbm, idx_v, rows_v, sem):
        wid = lax.axis_index("s") * _NC + lax.axis_index("c")
        base = wid * _B_PER_W
        pltpu.sync_copy(idx_hbm.at[pl.ds(base, _B_PER_W)], idx_v)
        pltpu.async_copy(table_hbm.at[idx_v], rows_v, sem).wait()
        pltpu.sync_copy(rows_v, out_hbm.at[pl.ds(base, _B_PER_W)])

    return gather_kernel


_gather = _make_gather()


def kernel(t, pos_encoding):
    idx = t.reshape(BATCH).astype(jnp.int32)
    return _gather(pos_encoding, idx)
